# baseline (device time: 305790 ns/iter reference)
import jax
import jax.numpy as jnp
from jax import lax
from jax.experimental import pallas as pl
from jax.experimental.pallas import tpu as pltpu

N_DEV = 16
N_Z = 4
N_W = 4


def _snap_e4m3(v):
    q = jnp.clip(v, -448.0, 448.0).astype(jnp.float8_e4m3fn)
    return q.astype(jnp.float32)


def kernel(x, w_mat):
    m_per, k = x.shape
    _, n_per = w_mat.shape
    half = m_per // 2

    def body(x_ref, w_ref, out_ref,
             up_buf, down_buf, stage, from_prev, from_next, anti, rz1,
             maxima_ref,
             up_send, up_recv, down_send, down_recv,
             h1n_send, h1p_send, fp_recv, fn_recv,
             ft_send, fb_send, at_recv, ab_recv,
             relay_send, relay_recv,
             cr_h1n, cr_h1p, cr_fwd_n, cr_fwd_p, cr_x,
             amax_send_sems, amax_recv_sems):
        my = lax.axis_index("i")
        z = my // N_W
        w = my % N_W
        nxt = z * N_W + lax.rem(w + 1, N_W)
        prv = z * N_W + lax.rem(w - 1 + N_W, N_W)
        anti_w = z * N_W + lax.rem(w + 2, N_W)
        up = my + N_W
        down = my - N_W

        barrier_sem = pltpu.get_barrier_semaphore()
        for nbr in (nxt, prv):
            pl.semaphore_signal(barrier_sem, inc=1, device_id=(nbr,),
                                device_id_type=pl.DeviceIdType.MESH)

        @pl.when(z < N_Z - 1)
        def _():
            pl.semaphore_signal(barrier_sem, inc=1, device_id=(up,),
                                device_id_type=pl.DeviceIdType.MESH)

        @pl.when(z > 0)
        def _():
            pl.semaphore_signal(barrier_sem, inc=1, device_id=(down,),
                                device_id_type=pl.DeviceIdType.MESH)

        n_nbrs = 2 + (z > 0).astype(jnp.int32) + (z < N_Z - 1).astype(jnp.int32)
        pl.semaphore_wait(barrier_sem, n_nbrs)

        def up_step(s):
            return pltpu.make_async_remote_copy(
                src_ref=x_ref if s == 0 else up_buf.at[s - 1],
                dst_ref=up_buf.at[s],
                send_sem=up_send.at[s],
                recv_sem=up_recv.at[s],
                device_id=(up,),
                device_id_type=pl.DeviceIdType.MESH,
            )

        def down_step(s):
            return pltpu.make_async_remote_copy(
                src_ref=x_ref if s == 0 else down_buf.at[s - 1],
                dst_ref=down_buf.at[s],
                send_sem=down_send.at[s],
                recv_sem=down_recv.at[s],
                device_id=(down,),
                device_id_type=pl.DeviceIdType.MESH,
            )

        def up_send_cond(s):
            return (z >= s) & (z < N_Z - 1)

        def down_send_cond(s):
            return (z <= N_Z - 1 - s) & (z > 0)

        @pl.when(up_send_cond(0))
        def _():
            up_step(0).start()

        @pl.when(down_send_cond(0))
        def _():
            down_step(0).start()

        def gemm(src):
            return jnp.dot(src, w_ref[:, :],
                           preferred_element_type=jnp.float32,
                           precision=lax.Precision.HIGHEST)

        def store(origin, y):
            out_ref[pl.ds(origin * m_per, m_per), :] = y

        def phase1_interleave(qq):
            s = qq - 1

            @pl.when(z >= s + 1)
            def _():
                up_step(s).wait_recv()

            @pl.when(z <= 2 - s)
            def _():
                down_step(s).wait_recv()

            if s + 1 <= 2:
                @pl.when(up_send_cond(s + 1))
                def _():
                    up_step(s + 1).start()

                @pl.when(down_send_cond(s + 1))
                def _():
                    down_step(s + 1).start()

        def stage_copy(qq):
            if qq == 1:
                @pl.when(z == 0)
                def _():
                    stage[0, :, :] = down_buf[0, :, :]

                @pl.when(z >= 1)
                def _():
                    stage[0, :, :] = up_buf[0, :, :]

                return jnp.where(z == 0, 1, -1).astype(jnp.int32)
            if qq == 2:
                @pl.when(z == 0)
                def _():
                    stage[0, :, :] = down_buf[1, :, :]

                @pl.when((z == 1) | (z == 2))
                def _():
                    stage[0, :, :] = down_buf[0, :, :]

                @pl.when(z == 3)
                def _():
                    stage[0, :, :] = up_buf[1, :, :]

                return jnp.where(z == 0, 2,
                                 jnp.where(z == 3, -2, 1)).astype(jnp.int32)
            @pl.when(z == 0)
            def _():
                stage[0, :, :] = down_buf[2, :, :]

            @pl.when(z == 1)
            def _():
                stage[0, :, :] = down_buf[1, :, :]

            @pl.when(z == 2)
            def _():
                stage[0, :, :] = up_buf[1, :, :]

            @pl.when(z == 3)
            def _():
                stage[0, :, :] = up_buf[2, :, :]

            return jnp.where(z == 0, 3,
                             jnp.where(z == 1, 2,
                                       jnp.where(z == 2, -2, -3))
                             ).astype(jnp.int32)

        def h1_pair(qq):
            sl = qq % 2
            r_n = pltpu.make_async_remote_copy(
                src_ref=stage.at[0], dst_ref=from_prev.at[sl],
                send_sem=h1n_send.at[sl], recv_sem=fp_recv.at[sl],
                device_id=(nxt,), device_id_type=pl.DeviceIdType.MESH)
            r_p = pltpu.make_async_remote_copy(
                src_ref=stage.at[0], dst_ref=from_next.at[sl],
                send_sem=h1p_send.at[sl], recv_sem=fn_recv.at[sl],
                device_id=(prv,), device_id_type=pl.DeviceIdType.MESH)
            return r_n, r_p

        def fwd_pair(qq):
            sl = qq % 2
            r_t = pltpu.make_async_remote_copy(
                src_ref=from_prev.at[sl, pl.ds(0, half)],
                dst_ref=anti.at[0, pl.ds(0, half)],
                send_sem=ft_send.at[0], recv_sem=at_recv.at[0],
                device_id=(nxt,), device_id_type=pl.DeviceIdType.MESH)
            r_b = pltpu.make_async_remote_copy(
                src_ref=from_next.at[sl, pl.ds(half, half)],
                dst_ref=anti.at[0, pl.ds(half, half)],
                send_sem=fb_send.at[0], recv_sem=ab_recv.at[0],
                device_id=(prv,), device_id_type=pl.DeviceIdType.MESH)
            return r_t, r_b

        def relay(idx, src, dst, dev):
            return pltpu.make_async_remote_copy(
                src_ref=src, dst_ref=dst,
                send_sem=relay_send.at[idx],
                recv_sem=relay_recv.at[0 if idx <= 1 else 1],
                device_id=(dev,), device_id_type=pl.DeviceIdType.MESH)

        def t1():
            return relay(0, anti.at[0], rz1.at[0], up)

        def t2():
            return relay(1, anti.at[0], rz1.at[0], down)

        def t3():
            return relay(2, anti.at[0], x_ref, down)

        def t4():
            return relay(3, anti.at[0], x_ref, down)

        def t5():
            return relay(4, x_ref, x_ref, down)

        def t6():
            return relay(5, rz1.at[0], x_ref, up)

        amax = jnp.float32(0.0)

        def maxup(a, y):
            return jnp.maximum(a, jnp.max(jnp.abs(y)))

        r_h1 = [None] * N_W
        r_fwd = {}
        dzs = [None] * N_W

        stage[0, :, :] = x_ref[:, :]
        dzs[0] = jnp.int32(0)
        r_h1[0] = h1_pair(0)
        r_h1[0][0].start()
        r_h1[0][1].start()
        y = gemm(stage[0, :, :])
        store(my, y)
        amax = maxup(amax, y)

        for q in range(N_W):
            sl = q % 2

            if q == 1:
                pt, pb = r_fwd[0]
                pt.wait_recv()
                pb.wait_recv()
                y = gemm(anti[0, :, :])
                store(anti_w + N_W * dzs[0], y)
                amax = maxup(amax, y)

                @pl.when(z < N_Z - 1)
                def _():
                    t1().start()

                @pl.when(z == 1)
                def _():
                    t2().start()

                @pl.when(z == 3)
                def _():
                    pl.semaphore_wait(cr_x, 1)
                    t3().start()

                @pl.when(z == 2)
                def _():
                    pl.semaphore_wait(cr_x, 1)
                    t4().start()

            if q == 2:
                t1().wait_recv()
                y = gemm(rz1[0, :, :])
                store(anti_w + N_W * dzs[1], y)
                amax = maxup(amax, y)

                @pl.when(z < N_Z - 1)
                def _():
                    t1().wait_send()

                @pl.when(z == 1)
                def _():
                    t2().wait_send()

                @pl.when(z == 3)
                def _():
                    t3().wait_send()

                @pl.when(z == 2)
                def _():
                    t4().wait_send()

                pl.semaphore_signal(cr_fwd_n, inc=1, device_id=(prv,),
                                    device_id_type=pl.DeviceIdType.MESH)
                pl.semaphore_signal(cr_fwd_p, inc=1, device_id=(nxt,),
                                    device_id_type=pl.DeviceIdType.MESH)

                @pl.when(z == 1)
                def _():
                    t5().wait_recv()
                    pl.semaphore_wait(cr_x, 1)
                    t5().start()

                @pl.when(z == 2)
                def _():
                    pl.semaphore_wait(cr_x, 1)
                    t6().start()

            if q == 3:
                @pl.when(z != 1)
                def _():
                    t5().wait_recv()
                y = gemm(x_ref[:, :])
                store(anti_w + N_W * dzs[2], y)
                amax = maxup(amax, y)

            rn, rp = r_h1[q]
            rn.wait_recv()
            rp.wait_recv()

            if q in (0, 3):
                if q == 3:
                    pl.semaphore_wait(cr_fwd_n, 1)
                    pl.semaphore_wait(cr_fwd_p, 1)
                r_fwd[q] = fwd_pair(q)
                r_fwd[q][0].start()
                r_fwd[q][1].start()

            y = gemm(from_prev[sl, :, :])
            store(prv + N_W * dzs[q], y)
            amax = maxup(amax, y)
            y = gemm(from_next[sl, :, :])
            store(nxt + N_W * dzs[q], y)
            amax = maxup(amax, y)

            if q in (0, 3):
                r_fwd[q][0].wait_send()
                r_fwd[q][1].wait_send()
            if q < 2:
                pl.semaphore_signal(cr_h1n, inc=1, device_id=(prv,),
                                    device_id_type=pl.DeviceIdType.MESH)
                pl.semaphore_signal(cr_h1p, inc=1, device_id=(nxt,),
                                    device_id_type=pl.DeviceIdType.MESH)
            rn.wait_send()
            rp.wait_send()

            if q == 0:
                @pl.when(up_send_cond(0))
                def _():
                    up_step(0).wait_send()

                @pl.when(down_send_cond(0))
                def _():
                    down_step(0).wait_send()

                xt = jnp.where(z == N_Z - 1, down, up)
                pl.semaphore_signal(cr_x, inc=1, device_id=(xt,),
                                    device_id_type=pl.DeviceIdType.MESH)

            if q < N_W - 1:
                phase1_interleave(q + 1)
                dzs[q + 1] = stage_copy(q + 1)
                if q + 1 >= 2:
                    pl.semaphore_wait(cr_h1n, 1)
                    pl.semaphore_wait(cr_h1p, 1)
                r_h1[q + 1] = h1_pair(q + 1)
                r_h1[q + 1][0].start()
                r_h1[q + 1][1].start()
                y = gemm(stage[0, :, :])
                store(my + N_W * dzs[q + 1], y)
                amax = maxup(amax, y)

        pt, pb = r_fwd[3]
        pt.wait_recv()
        pb.wait_recv()
        y = gemm(anti[0, :, :])
        store(anti_w + N_W * dzs[3], y)
        amax = maxup(amax, y)

        for s in (1, 2):
            @pl.when(up_send_cond(s))
            def _(s=s):
                up_step(s).wait_send()

            @pl.when(down_send_cond(s))
            def _(s=s):
                down_step(s).wait_send()

        @pl.when(z == 1)
        def _():
            t5().wait_send()

        @pl.when(z == 2)
        def _():
            t6().wait_send()

        maxima_ref[pl.ds(my, 1), :] = jnp.full((1, 128), amax, jnp.float32)
        amax_rdmas = []
        for j in range(1, N_DEV):
            tgt = lax.rem(my + j, N_DEV)
            r = pltpu.make_async_remote_copy(
                src_ref=maxima_ref.at[pl.ds(my, 1)],
                dst_ref=maxima_ref.at[pl.ds(my, 1)],
                send_sem=amax_send_sems.at[j - 1],
                recv_sem=amax_recv_sems.at[j - 1],
                device_id=(tgt,),
                device_id_type=pl.DeviceIdType.MESH,
            )
            r.start()
            amax_rdmas.append(r)
        for r in amax_rdmas:
            r.wait_send()
        for r in amax_rdmas:
            r.wait_recv()

        gmax = jnp.max(maxima_ref[:, :])
        scale = gmax / 448.0
        out_ref[:, :] = _snap_e4m3(out_ref[:, :] / scale) * scale

    return pl.pallas_call(
        body,
        out_shape=jax.ShapeDtypeStruct((N_DEV * m_per, n_per), jnp.float32),
        in_specs=[
            pl.BlockSpec(memory_space=pltpu.VMEM),
            pl.BlockSpec(memory_space=pltpu.VMEM),
        ],
        out_specs=pl.BlockSpec(memory_space=pltpu.VMEM),
        scratch_shapes=[
            pltpu.VMEM((3, m_per, k), jnp.float32),
            pltpu.VMEM((3, m_per, k), jnp.float32),
            pltpu.VMEM((1, m_per, k), jnp.float32),
            pltpu.VMEM((2, m_per, k), jnp.float32),
            pltpu.VMEM((2, m_per, k), jnp.float32),
            pltpu.VMEM((1, m_per, k), jnp.float32),
            pltpu.VMEM((1, m_per, k), jnp.float32),
            pltpu.VMEM((N_DEV, 128), jnp.float32),
            pltpu.SemaphoreType.DMA((3,)),
            pltpu.SemaphoreType.DMA((3,)),
            pltpu.SemaphoreType.DMA((3,)),
            pltpu.SemaphoreType.DMA((3,)),
            pltpu.SemaphoreType.DMA((2,)),
            pltpu.SemaphoreType.DMA((2,)),
            pltpu.SemaphoreType.DMA((2,)),
            pltpu.SemaphoreType.DMA((2,)),
            pltpu.SemaphoreType.DMA((1,)),
            pltpu.SemaphoreType.DMA((1,)),
            pltpu.SemaphoreType.DMA((1,)),
            pltpu.SemaphoreType.DMA((1,)),
            pltpu.SemaphoreType.DMA((6,)),
            pltpu.SemaphoreType.DMA((2,)),
            pltpu.SemaphoreType.REGULAR,
            pltpu.SemaphoreType.REGULAR,
            pltpu.SemaphoreType.REGULAR,
            pltpu.SemaphoreType.REGULAR,
            pltpu.SemaphoreType.REGULAR,
            pltpu.SemaphoreType.DMA((N_DEV - 1,)),
            pltpu.SemaphoreType.DMA((N_DEV - 1,)),
        ],
        compiler_params=pltpu.CompilerParams(
            collective_id=0, vmem_limit_bytes=100 * 1024 * 1024),
    )(x, w_mat)


# device time: 300780 ns/iter; 1.0167x vs baseline; 1.0167x over previous
import jax
import jax.numpy as jnp
from jax import lax
from jax.experimental import pallas as pl
from jax.experimental.pallas import tpu as pltpu

N_DEV = 16
N_Z = 4
N_W = 4


def _snap_e4m3(v):
    q = jnp.clip(v, -448.0, 448.0).astype(jnp.float8_e4m3fn)
    return q.astype(jnp.float32)


def kernel(x, w_mat):
    m_per, k = x.shape
    _, n_per = w_mat.shape
    half = m_per // 2

    def body(x_ref, w_ref, out_ref,
             up_buf, down_buf, stage, from_prev, from_next, anti, rz1,
             maxima_ref,
             up_send, up_recv, down_send, down_recv,
             h1n_send, h1p_send, fp_recv, fn_recv,
             ft_send, fb_send, at_recv, ab_recv,
             relay_send, relay_recv,
             cr_h1n, cr_h1p, cr_fwd_n, cr_fwd_p,
             amax_send_sems, amax_recv_sems):
        my = lax.axis_index("i")
        z = my // N_W
        w = my % N_W
        nxt = z * N_W + lax.rem(w + 1, N_W)
        prv = z * N_W + lax.rem(w - 1 + N_W, N_W)
        anti_w = z * N_W + lax.rem(w + 2, N_W)
        up = my + N_W
        down = my - N_W

        barrier_sem = pltpu.get_barrier_semaphore()
        for nbr in (nxt, prv):
            pl.semaphore_signal(barrier_sem, inc=1, device_id=(nbr,),
                                device_id_type=pl.DeviceIdType.MESH)

        @pl.when(z < N_Z - 1)
        def _():
            pl.semaphore_signal(barrier_sem, inc=1, device_id=(up,),
                                device_id_type=pl.DeviceIdType.MESH)

        @pl.when(z > 0)
        def _():
            pl.semaphore_signal(barrier_sem, inc=1, device_id=(down,),
                                device_id_type=pl.DeviceIdType.MESH)

        n_nbrs = 2 + (z > 0).astype(jnp.int32) + (z < N_Z - 1).astype(jnp.int32)
        pl.semaphore_wait(barrier_sem, n_nbrs)

        def up_step(s):
            return pltpu.make_async_remote_copy(
                src_ref=x_ref if s == 0 else up_buf.at[s - 1],
                dst_ref=up_buf.at[s],
                send_sem=up_send.at[s],
                recv_sem=up_recv.at[s],
                device_id=(up,),
                device_id_type=pl.DeviceIdType.MESH,
            )

        def down_step(s):
            return pltpu.make_async_remote_copy(
                src_ref=x_ref if s == 0 else down_buf.at[s - 1],
                dst_ref=down_buf.at[s],
                send_sem=down_send.at[s],
                recv_sem=down_recv.at[s],
                device_id=(down,),
                device_id_type=pl.DeviceIdType.MESH,
            )

        def up_send_cond(s):
            return (z >= s) & (z < N_Z - 1)

        def down_send_cond(s):
            return (z <= N_Z - 1 - s) & (z > 0)

        @pl.when(up_send_cond(0))
        def _():
            up_step(0).start()

        @pl.when(down_send_cond(0))
        def _():
            down_step(0).start()

        def gemm(src):
            return jnp.dot(src, w_ref[:, :],
                           preferred_element_type=jnp.float32,
                           precision=lax.Precision.HIGHEST)

        def store(origin, y):
            out_ref[pl.ds(origin * m_per, m_per), :] = y

        def phase1_interleave(qq):
            s = qq - 1

            @pl.when(z >= s + 1)
            def _():
                up_step(s).wait_recv()

            @pl.when(z <= 2 - s)
            def _():
                down_step(s).wait_recv()

            if s + 1 <= 2:
                @pl.when(up_send_cond(s + 1))
                def _():
                    up_step(s + 1).start()

                @pl.when(down_send_cond(s + 1))
                def _():
                    down_step(s + 1).start()

        def stage_copy(qq):
            if qq == 1:
                @pl.when(z == 0)
                def _():
                    stage[0, :, :] = down_buf[0, :, :]

                @pl.when(z >= 1)
                def _():
                    stage[0, :, :] = up_buf[0, :, :]

                return jnp.where(z == 0, 1, -1).astype(jnp.int32)
            if qq == 2:
                @pl.when(z == 0)
                def _():
                    stage[0, :, :] = down_buf[1, :, :]

                @pl.when((z == 1) | (z == 2))
                def _():
                    stage[0, :, :] = down_buf[0, :, :]

                @pl.when(z == 3)
                def _():
                    stage[0, :, :] = up_buf[1, :, :]

                return jnp.where(z == 0, 2,
                                 jnp.where(z == 3, -2, 1)).astype(jnp.int32)
            @pl.when(z == 0)
            def _():
                stage[0, :, :] = down_buf[2, :, :]

            @pl.when(z == 1)
            def _():
                stage[0, :, :] = down_buf[1, :, :]

            @pl.when(z == 2)
            def _():
                stage[0, :, :] = up_buf[1, :, :]

            @pl.when(z == 3)
            def _():
                stage[0, :, :] = up_buf[2, :, :]

            return jnp.where(z == 0, 3,
                             jnp.where(z == 1, 2,
                                       jnp.where(z == 2, -2, -3))
                             ).astype(jnp.int32)

        def h1_pair(qq):
            sl = qq % 2
            r_n = pltpu.make_async_remote_copy(
                src_ref=stage.at[0], dst_ref=from_prev.at[sl],
                send_sem=h1n_send.at[sl], recv_sem=fp_recv.at[sl],
                device_id=(nxt,), device_id_type=pl.DeviceIdType.MESH)
            r_p = pltpu.make_async_remote_copy(
                src_ref=stage.at[0], dst_ref=from_next.at[sl],
                send_sem=h1p_send.at[sl], recv_sem=fn_recv.at[sl],
                device_id=(prv,), device_id_type=pl.DeviceIdType.MESH)
            return r_n, r_p

        def fwd_pair(qq):
            sl = qq % 2
            r_t = pltpu.make_async_remote_copy(
                src_ref=from_prev.at[sl, pl.ds(0, half)],
                dst_ref=anti.at[0, pl.ds(0, half)],
                send_sem=ft_send.at[0], recv_sem=at_recv.at[0],
                device_id=(nxt,), device_id_type=pl.DeviceIdType.MESH)
            r_b = pltpu.make_async_remote_copy(
                src_ref=from_next.at[sl, pl.ds(half, half)],
                dst_ref=anti.at[0, pl.ds(half, half)],
                send_sem=fb_send.at[0], recv_sem=ab_recv.at[0],
                device_id=(prv,), device_id_type=pl.DeviceIdType.MESH)
            return r_t, r_b

        def relay(idx, src, dst, dev):
            return pltpu.make_async_remote_copy(
                src_ref=src, dst_ref=dst,
                send_sem=relay_send.at[idx],
                recv_sem=relay_recv.at[0],
                device_id=(dev,), device_id_type=pl.DeviceIdType.MESH)

        def t1():
            return relay(0, anti.at[0], rz1.at[0], up)

        def t2():
            return relay(1, anti.at[0], rz1.at[0], down)

        amax = jnp.float32(0.0)

        def maxup(a, y):
            return jnp.maximum(a, jnp.max(jnp.abs(y)))

        r_h1 = [None] * N_W
        r_fwd = {}
        dzs = [None] * N_W

        stage[0, :, :] = x_ref[:, :]
        dzs[0] = jnp.int32(0)
        r_h1[0] = h1_pair(0)
        r_h1[0][0].start()
        r_h1[0][1].start()
        y = gemm(stage[0, :, :])
        store(my, y)
        amax = maxup(amax, y)

        for q in range(N_W):
            sl = q % 2

            if q == 1:
                pt, pb = r_fwd[0]
                pt.wait_recv()
                pb.wait_recv()
                y = gemm(anti[0, :, :])
                store(anti_w + N_W * dzs[0], y)
                amax = maxup(amax, y)

                @pl.when(z < N_Z - 1)
                def _():
                    t1().start()

                @pl.when(z == 1)
                def _():
                    t2().start()

            if q == 2:
                t1().wait_recv()
                y = gemm(rz1[0, :, :])
                store(anti_w + N_W * dzs[1], y)
                amax = maxup(amax, y)

                @pl.when(z < N_Z - 1)
                def _():
                    t1().wait_send()

                @pl.when(z == 1)
                def _():
                    t2().wait_send()

                pl.semaphore_signal(cr_fwd_n, inc=1, device_id=(prv,),
                                    device_id_type=pl.DeviceIdType.MESH)
                pl.semaphore_signal(cr_fwd_p, inc=1, device_id=(nxt,),
                                    device_id_type=pl.DeviceIdType.MESH)

            if q == 3:
                pt, pb = r_fwd[2]
                pt.wait_recv()
                pb.wait_recv()
                y = gemm(anti[0, :, :])
                store(anti_w + N_W * dzs[2], y)
                amax = maxup(amax, y)
                pl.semaphore_signal(cr_fwd_n, inc=1, device_id=(prv,),
                                    device_id_type=pl.DeviceIdType.MESH)
                pl.semaphore_signal(cr_fwd_p, inc=1, device_id=(nxt,),
                                    device_id_type=pl.DeviceIdType.MESH)

            rn, rp = r_h1[q]
            rn.wait_recv()
            rp.wait_recv()

            if q in (0, 2, 3):
                if q >= 2:
                    pl.semaphore_wait(cr_fwd_n, 1)
                    pl.semaphore_wait(cr_fwd_p, 1)
                r_fwd[q] = fwd_pair(q)
                r_fwd[q][0].start()
                r_fwd[q][1].start()

            y = gemm(from_prev[sl, :, :])
            store(prv + N_W * dzs[q], y)
            amax = maxup(amax, y)
            y = gemm(from_next[sl, :, :])
            store(nxt + N_W * dzs[q], y)
            amax = maxup(amax, y)

            if q in (0, 2, 3):
                r_fwd[q][0].wait_send()
                r_fwd[q][1].wait_send()
            if q < 2:
                pl.semaphore_signal(cr_h1n, inc=1, device_id=(prv,),
                                    device_id_type=pl.DeviceIdType.MESH)
                pl.semaphore_signal(cr_h1p, inc=1, device_id=(nxt,),
                                    device_id_type=pl.DeviceIdType.MESH)
            rn.wait_send()
            rp.wait_send()

            if q == 0:
                @pl.when(up_send_cond(0))
                def _():
                    up_step(0).wait_send()

                @pl.when(down_send_cond(0))
                def _():
                    down_step(0).wait_send()

            if q < N_W - 1:
                phase1_interleave(q + 1)
                dzs[q + 1] = stage_copy(q + 1)
                if q + 1 >= 2:
                    pl.semaphore_wait(cr_h1n, 1)
                    pl.semaphore_wait(cr_h1p, 1)
                r_h1[q + 1] = h1_pair(q + 1)
                r_h1[q + 1][0].start()
                r_h1[q + 1][1].start()
                y = gemm(stage[0, :, :])
                store(my + N_W * dzs[q + 1], y)
                amax = maxup(amax, y)

        pt, pb = r_fwd[3]
        pt.wait_recv()
        pb.wait_recv()
        y = gemm(anti[0, :, :])
        store(anti_w + N_W * dzs[3], y)
        amax = maxup(amax, y)

        for s in (1, 2):
            @pl.when(up_send_cond(s))
            def _(s=s):
                up_step(s).wait_send()

            @pl.when(down_send_cond(s))
            def _(s=s):
                down_step(s).wait_send()

        maxima_ref[pl.ds(my, 1), :] = jnp.full((1, 128), amax, jnp.float32)
        amax_rdmas = []
        for j in range(1, N_DEV):
            tgt = lax.rem(my + j, N_DEV)
            r = pltpu.make_async_remote_copy(
                src_ref=maxima_ref.at[pl.ds(my, 1)],
                dst_ref=maxima_ref.at[pl.ds(my, 1)],
                send_sem=amax_send_sems.at[j - 1],
                recv_sem=amax_recv_sems.at[j - 1],
                device_id=(tgt,),
                device_id_type=pl.DeviceIdType.MESH,
            )
            r.start()
            amax_rdmas.append(r)
        for r in amax_rdmas:
            r.wait_send()
        for r in amax_rdmas:
            r.wait_recv()

        gmax = jnp.max(maxima_ref[:, :])
        scale = gmax / 448.0
        out_ref[:, :] = _snap_e4m3(out_ref[:, :] / scale) * scale

    return pl.pallas_call(
        body,
        out_shape=jax.ShapeDtypeStruct((N_DEV * m_per, n_per), jnp.float32),
        in_specs=[
            pl.BlockSpec(memory_space=pltpu.VMEM),
            pl.BlockSpec(memory_space=pltpu.VMEM),
        ],
        out_specs=pl.BlockSpec(memory_space=pltpu.VMEM),
        scratch_shapes=[
            pltpu.VMEM((3, m_per, k), jnp.float32),
            pltpu.VMEM((3, m_per, k), jnp.float32),
            pltpu.VMEM((1, m_per, k), jnp.float32),
            pltpu.VMEM((2, m_per, k), jnp.float32),
            pltpu.VMEM((2, m_per, k), jnp.float32),
            pltpu.VMEM((1, m_per, k), jnp.float32),
            pltpu.VMEM((1, m_per, k), jnp.float32),
            pltpu.VMEM((N_DEV, 128), jnp.float32),
            pltpu.SemaphoreType.DMA((3,)),
            pltpu.SemaphoreType.DMA((3,)),
            pltpu.SemaphoreType.DMA((3,)),
            pltpu.SemaphoreType.DMA((3,)),
            pltpu.SemaphoreType.DMA((2,)),
            pltpu.SemaphoreType.DMA((2,)),
            pltpu.SemaphoreType.DMA((2,)),
            pltpu.SemaphoreType.DMA((2,)),
            pltpu.SemaphoreType.DMA((1,)),
            pltpu.SemaphoreType.DMA((1,)),
            pltpu.SemaphoreType.DMA((1,)),
            pltpu.SemaphoreType.DMA((1,)),
            pltpu.SemaphoreType.DMA((2,)),
            pltpu.SemaphoreType.DMA((1,)),
            pltpu.SemaphoreType.REGULAR,
            pltpu.SemaphoreType.REGULAR,
            pltpu.SemaphoreType.REGULAR,
            pltpu.SemaphoreType.REGULAR,
            pltpu.SemaphoreType.DMA((N_DEV - 1,)),
            pltpu.SemaphoreType.DMA((N_DEV - 1,)),
        ],
        compiler_params=pltpu.CompilerParams(
            collective_id=0, vmem_limit_bytes=100 * 1024 * 1024),
    )(x, w_mat)


# device time: 296889 ns/iter; 1.0300x vs baseline; 1.0131x over previous
import jax
import jax.numpy as jnp
from jax import lax
from jax.experimental import pallas as pl
from jax.experimental.pallas import tpu as pltpu

N_DEV = 16
N_Z = 4
N_W = 4


def _snap_e4m3(v):
    q = jnp.clip(v, -448.0, 448.0).astype(jnp.float8_e4m3fn)
    return q.astype(jnp.float32)


def kernel(x, w_mat):
    m_per, k = x.shape
    _, n_per = w_mat.shape
    half = m_per // 2

    def body(x_ref, w_ref, out_ref,
             up_buf, down_buf, stage, from_prev, from_next, anti, rz1,
             maxima_ref,
             up_send, up_recv, down_send, down_recv,
             h1n_send, h1p_send, fp_recv, fn_recv,
             ft_send, fb_send, at_recv, ab_recv,
             relay_send, relay_recv,
             cr_h1n, cr_h1p, cr_fwd_n, cr_fwd_p,
             amax_send_sems, amax_recv_sems):
        my = lax.axis_index("i")
        z = my // N_W
        w = my % N_W
        nxt = z * N_W + lax.rem(w + 1, N_W)
        prv = z * N_W + lax.rem(w - 1 + N_W, N_W)
        anti_w = z * N_W + lax.rem(w + 2, N_W)
        up = my + N_W
        down = my - N_W

        barrier_sem = pltpu.get_barrier_semaphore()
        for nbr in (nxt, prv):
            pl.semaphore_signal(barrier_sem, inc=1, device_id=(nbr,),
                                device_id_type=pl.DeviceIdType.MESH)

        @pl.when(z < N_Z - 1)
        def _():
            pl.semaphore_signal(barrier_sem, inc=1, device_id=(up,),
                                device_id_type=pl.DeviceIdType.MESH)

        @pl.when(z > 0)
        def _():
            pl.semaphore_signal(barrier_sem, inc=1, device_id=(down,),
                                device_id_type=pl.DeviceIdType.MESH)

        n_nbrs = 2 + (z > 0).astype(jnp.int32) + (z < N_Z - 1).astype(jnp.int32)
        pl.semaphore_wait(barrier_sem, n_nbrs)

        def up_step(s):
            return pltpu.make_async_remote_copy(
                src_ref=x_ref if s == 0 else up_buf.at[s - 1],
                dst_ref=up_buf.at[s],
                send_sem=up_send.at[s],
                recv_sem=up_recv.at[s],
                device_id=(up,),
                device_id_type=pl.DeviceIdType.MESH,
            )

        def down_step(s):
            return pltpu.make_async_remote_copy(
                src_ref=x_ref if s == 0 else down_buf.at[s - 1],
                dst_ref=down_buf.at[s],
                send_sem=down_send.at[s],
                recv_sem=down_recv.at[s],
                device_id=(down,),
                device_id_type=pl.DeviceIdType.MESH,
            )

        def up_send_cond(s):
            return (z >= s) & (z < N_Z - 1)

        def down_send_cond(s):
            return (z <= N_Z - 1 - s) & (z > 0)

        @pl.when(up_send_cond(0))
        def _():
            up_step(0).start()

        @pl.when(down_send_cond(0))
        def _():
            down_step(0).start()

        def gemm(src):
            return jnp.dot(src, w_ref[:, :],
                           preferred_element_type=jnp.float32,
                           precision=lax.Precision.HIGHEST)

        def store(origin, y):
            out_ref[pl.ds(origin * m_per, m_per), :] = y

        def phase1_interleave(qq):
            s = qq - 1

            @pl.when(z >= s + 1)
            def _():
                up_step(s).wait_recv()

            @pl.when(z <= 2 - s)
            def _():
                down_step(s).wait_recv()

            if s + 1 <= 2:
                @pl.when(up_send_cond(s + 1))
                def _():
                    up_step(s + 1).start()

                @pl.when(down_send_cond(s + 1))
                def _():
                    down_step(s + 1).start()

        def stage_copy(qq):
            if qq == 1:
                @pl.when(z == 0)
                def _():
                    stage[0, :, :] = down_buf[0, :, :]

                @pl.when(z >= 1)
                def _():
                    stage[0, :, :] = up_buf[0, :, :]

                return jnp.where(z == 0, 1, -1).astype(jnp.int32)
            if qq == 2:
                @pl.when(z == 0)
                def _():
                    stage[0, :, :] = down_buf[1, :, :]

                @pl.when((z == 1) | (z == 2))
                def _():
                    stage[0, :, :] = down_buf[0, :, :]

                @pl.when(z == 3)
                def _():
                    stage[0, :, :] = up_buf[1, :, :]

                return jnp.where(z == 0, 2,
                                 jnp.where(z == 3, -2, 1)).astype(jnp.int32)
            @pl.when(z == 0)
            def _():
                stage[0, :, :] = down_buf[2, :, :]

            @pl.when(z == 1)
            def _():
                stage[0, :, :] = down_buf[1, :, :]

            @pl.when(z == 2)
            def _():
                stage[0, :, :] = up_buf[1, :, :]

            @pl.when(z == 3)
            def _():
                stage[0, :, :] = up_buf[2, :, :]

            return jnp.where(z == 0, 3,
                             jnp.where(z == 1, 2,
                                       jnp.where(z == 2, -2, -3))
                             ).astype(jnp.int32)

        def h1_pair(qq):
            sl = qq % 2
            r_n = pltpu.make_async_remote_copy(
                src_ref=stage.at[0], dst_ref=from_prev.at[sl],
                send_sem=h1n_send.at[sl], recv_sem=fp_recv.at[sl],
                device_id=(nxt,), device_id_type=pl.DeviceIdType.MESH)
            r_p = pltpu.make_async_remote_copy(
                src_ref=stage.at[0], dst_ref=from_next.at[sl],
                send_sem=h1p_send.at[sl], recv_sem=fn_recv.at[sl],
                device_id=(prv,), device_id_type=pl.DeviceIdType.MESH)
            return r_n, r_p

        def fwd_pair(qq):
            sl = qq % 2
            r_t = pltpu.make_async_remote_copy(
                src_ref=from_prev.at[sl, pl.ds(0, half)],
                dst_ref=anti.at[0, pl.ds(0, half)],
                send_sem=ft_send.at[0], recv_sem=at_recv.at[0],
                device_id=(nxt,), device_id_type=pl.DeviceIdType.MESH)
            r_b = pltpu.make_async_remote_copy(
                src_ref=from_next.at[sl, pl.ds(half, half)],
                dst_ref=anti.at[0, pl.ds(half, half)],
                send_sem=fb_send.at[0], recv_sem=ab_recv.at[0],
                device_id=(prv,), device_id_type=pl.DeviceIdType.MESH)
            return r_t, r_b

        def relay(idx, src, dst, dev):
            return pltpu.make_async_remote_copy(
                src_ref=src, dst_ref=dst,
                send_sem=relay_send.at[idx],
                recv_sem=relay_recv.at[0],
                device_id=(dev,), device_id_type=pl.DeviceIdType.MESH)

        def t1():
            return relay(0, anti.at[0], rz1.at[0], up)

        def t2():
            return relay(1, anti.at[0], rz1.at[0], down)

        amax = jnp.float32(0.0)

        def maxup(a, y):
            return jnp.maximum(a, jnp.max(jnp.abs(y)))

        r_h1 = [None] * N_W
        r_fwd = {}
        dzs = [None] * N_W

        stage[0, :, :] = x_ref[:, :]
        dzs[0] = jnp.int32(0)
        r_h1[0] = h1_pair(0)
        r_h1[0][0].start()
        r_h1[0][1].start()
        y = gemm(stage[0, :, :])
        store(my, y)
        amax = maxup(amax, y)

        for q in range(N_W):
            sl = q % 2

            if q == 1:
                pt, pb = r_fwd[0]
                pt.wait_recv()
                pb.wait_recv()
                y = gemm(anti[0, :, :])
                store(anti_w + N_W * dzs[0], y)
                amax = maxup(amax, y)

                @pl.when(z < N_Z - 1)
                def _():
                    t1().start()

                @pl.when(z == 1)
                def _():
                    t2().start()

            if q == 2:
                t1().wait_recv()
                y = gemm(rz1[0, :, :])
                store(anti_w + N_W * dzs[1], y)
                amax = maxup(amax, y)

                @pl.when(z < N_Z - 1)
                def _():
                    t1().wait_send()

                @pl.when(z == 1)
                def _():
                    t2().wait_send()

                pl.semaphore_signal(cr_fwd_n, inc=1, device_id=(prv,),
                                    device_id_type=pl.DeviceIdType.MESH)
                pl.semaphore_signal(cr_fwd_p, inc=1, device_id=(nxt,),
                                    device_id_type=pl.DeviceIdType.MESH)

            if q == 3:
                pt, pb = r_fwd[2]
                pt.wait_recv()
                pb.wait_recv()
                y = gemm(anti[0, :, :])
                store(anti_w + N_W * dzs[2], y)
                amax = maxup(amax, y)
                pl.semaphore_signal(cr_fwd_n, inc=1, device_id=(prv,),
                                    device_id_type=pl.DeviceIdType.MESH)
                pl.semaphore_signal(cr_fwd_p, inc=1, device_id=(nxt,),
                                    device_id_type=pl.DeviceIdType.MESH)

            rn, rp = r_h1[q]
            rn.wait_recv()
            rp.wait_recv()

            if q in (0, 2, 3):
                if q >= 2:
                    pl.semaphore_wait(cr_fwd_n, 1)
                    pl.semaphore_wait(cr_fwd_p, 1)
                r_fwd[q] = fwd_pair(q)
                r_fwd[q][0].start()
                r_fwd[q][1].start()

            y = gemm(from_prev[sl, :, :])
            store(prv + N_W * dzs[q], y)
            amax = maxup(amax, y)
            y = gemm(from_next[sl, :, :])
            store(nxt + N_W * dzs[q], y)
            amax = maxup(amax, y)

            rn.wait_send()
            rp.wait_send()

            if q < N_W - 1:
                phase1_interleave(q + 1)
                dzs[q + 1] = stage_copy(q + 1)
                if q + 1 >= 2:
                    pl.semaphore_wait(cr_h1n, 1)
                    pl.semaphore_wait(cr_h1p, 1)
                r_h1[q + 1] = h1_pair(q + 1)
                r_h1[q + 1][0].start()
                r_h1[q + 1][1].start()
                y = gemm(stage[0, :, :])
                store(my + N_W * dzs[q + 1], y)
                amax = maxup(amax, y)

            if q in (0, 2, 3):
                r_fwd[q][0].wait_send()
                r_fwd[q][1].wait_send()
            if q < 2:
                pl.semaphore_signal(cr_h1n, inc=1, device_id=(prv,),
                                    device_id_type=pl.DeviceIdType.MESH)
                pl.semaphore_signal(cr_h1p, inc=1, device_id=(nxt,),
                                    device_id_type=pl.DeviceIdType.MESH)

            if q == 0:
                @pl.when(up_send_cond(0))
                def _():
                    up_step(0).wait_send()

                @pl.when(down_send_cond(0))
                def _():
                    down_step(0).wait_send()

        pt, pb = r_fwd[3]
        pt.wait_recv()
        pb.wait_recv()
        y = gemm(anti[0, :, :])
        store(anti_w + N_W * dzs[3], y)
        amax = maxup(amax, y)

        for s in (1, 2):
            @pl.when(up_send_cond(s))
            def _(s=s):
                up_step(s).wait_send()

            @pl.when(down_send_cond(s))
            def _(s=s):
                down_step(s).wait_send()

        maxima_ref[pl.ds(my, 1), :] = jnp.full((1, 128), amax, jnp.float32)
        amax_rdmas = []
        for j in range(1, N_DEV):
            tgt = lax.rem(my + j, N_DEV)
            r = pltpu.make_async_remote_copy(
                src_ref=maxima_ref.at[pl.ds(my, 1)],
                dst_ref=maxima_ref.at[pl.ds(my, 1)],
                send_sem=amax_send_sems.at[j - 1],
                recv_sem=amax_recv_sems.at[j - 1],
                device_id=(tgt,),
                device_id_type=pl.DeviceIdType.MESH,
            )
            r.start()
            amax_rdmas.append(r)
        for r in amax_rdmas:
            r.wait_send()
        for r in amax_rdmas:
            r.wait_recv()

        gmax = jnp.max(maxima_ref[:, :])
        scale = gmax / 448.0
        out_ref[:, :] = _snap_e4m3(out_ref[:, :] / scale) * scale

    return pl.pallas_call(
        body,
        out_shape=jax.ShapeDtypeStruct((N_DEV * m_per, n_per), jnp.float32),
        in_specs=[
            pl.BlockSpec(memory_space=pltpu.VMEM),
            pl.BlockSpec(memory_space=pltpu.VMEM),
        ],
        out_specs=pl.BlockSpec(memory_space=pltpu.VMEM),
        scratch_shapes=[
            pltpu.VMEM((3, m_per, k), jnp.float32),
            pltpu.VMEM((3, m_per, k), jnp.float32),
            pltpu.VMEM((1, m_per, k), jnp.float32),
            pltpu.VMEM((2, m_per, k), jnp.float32),
            pltpu.VMEM((2, m_per, k), jnp.float32),
            pltpu.VMEM((1, m_per, k), jnp.float32),
            pltpu.VMEM((1, m_per, k), jnp.float32),
            pltpu.VMEM((N_DEV, 128), jnp.float32),
            pltpu.SemaphoreType.DMA((3,)),
            pltpu.SemaphoreType.DMA((3,)),
            pltpu.SemaphoreType.DMA((3,)),
            pltpu.SemaphoreType.DMA((3,)),
            pltpu.SemaphoreType.DMA((2,)),
            pltpu.SemaphoreType.DMA((2,)),
            pltpu.SemaphoreType.DMA((2,)),
            pltpu.SemaphoreType.DMA((2,)),
            pltpu.SemaphoreType.DMA((1,)),
            pltpu.SemaphoreType.DMA((1,)),
            pltpu.SemaphoreType.DMA((1,)),
            pltpu.SemaphoreType.DMA((1,)),
            pltpu.SemaphoreType.DMA((2,)),
            pltpu.SemaphoreType.DMA((1,)),
            pltpu.SemaphoreType.REGULAR,
            pltpu.SemaphoreType.REGULAR,
            pltpu.SemaphoreType.REGULAR,
            pltpu.SemaphoreType.REGULAR,
            pltpu.SemaphoreType.DMA((N_DEV - 1,)),
            pltpu.SemaphoreType.DMA((N_DEV - 1,)),
        ],
        compiler_params=pltpu.CompilerParams(
            collective_id=0, vmem_limit_bytes=100 * 1024 * 1024),
    )(x, w_mat)
